# baseline (speedup 1.0000x reference)
import functools

import jax
import jax.numpy as jnp
from jax import lax
from jax.experimental import pallas as pl
from jax.experimental.pallas import tpu as pltpu
from jax.experimental.pallas import tpu_sc as plsc

HEIGHT = 10
D = 2048
N_NODES = 2046
NP = 2048
BM = 256
BATCH = 4096
NW = 32
BPW = BATCH // NW
L = 16


def _scores_kernel(x_ref, w_ref, xi_ref, c_ref):
    f = jax.lax.dot_general(
        x_ref[...].astype(jnp.bfloat16), w_ref[...],
        (((1,), (0,)), ((), ())),
        preferred_element_type=jnp.float32)
    fb = f.astype(jnp.bfloat16)
    s = jax.lax.dot_general(
        fb, xi_ref[...], (((1,), (0,)), ((), ())),
        preferred_element_type=jnp.float32)
    r = jnp.concatenate([s[:, 1:], s[:, :1]], axis=1)
    c_ref[...] = (s >= r).astype(jnp.int32)


_SC_MESH = plsc.VectorSubcoreMesh(core_axis_name="c", subcore_axis_name="s")


@functools.partial(
    pl.kernel,
    mesh=_SC_MESH,
    out_type=jax.ShapeDtypeStruct((NW, HEIGHT, BPW), jnp.int32),
    scratch_types=[
        pltpu.VMEM((BPW,), jnp.int32),
        pltpu.VMEM((BPW,), jnp.int32),
        pltpu.VMEM((BPW,), jnp.int32),
        pltpu.VMEM((BPW,), jnp.int32),
        pltpu.VMEM((BPW,), jnp.int32),
        pltpu.SemaphoreType.DMA,
    ],
)
def _sc_descend(c_hbm, out_hbm, rowbase_v, idx_v, col_v, g_v, row_v, sem):
    wid = lax.axis_index("s") * 2 + lax.axis_index("c")
    base = wid * BPW

    @pl.loop(0, BPW, step=L)
    def _init(j):
        sl = pl.ds(j, L)
        lanes = lax.iota(jnp.int32, L)
        rowbase_v[sl] = (lanes + (base + j)) * NP
        idx_v[sl] = (lanes + (base + j)) * NP
        col_v[sl] = jnp.zeros((L,), jnp.int32)

    for h in range(HEIGHT):
        pltpu.async_copy(c_hbm.at[idx_v], g_v, sem).wait()

        @pl.loop(0, BPW, step=L)
        def _step(j):
            sl = pl.ds(j, L)
            nxt = col_v[sl] + 2 - g_v[sl]
            row_v[sl] = nxt
            col2 = 2 * nxt
            col_v[sl] = col2
            idx_v[sl] = rowbase_v[sl] + col2

        pltpu.sync_copy(row_v, out_hbm.at[wid, h])


def kernel(X, W, Xi):
    batch = X.shape[0]
    wb = W.astype(jnp.bfloat16)
    xib = jnp.pad(Xi.astype(jnp.bfloat16), ((0, 0), (0, NP - N_NODES)))

    c = pl.pallas_call(
        _scores_kernel,
        grid=(batch // BM,),
        in_specs=[
            pl.BlockSpec((BM, D), lambda i: (i, 0)),
            pl.BlockSpec((D, D), lambda i: (0, 0)),
            pl.BlockSpec((D, NP), lambda i: (0, 0)),
        ],
        out_specs=pl.BlockSpec((BM, NP), lambda i: (i, 0)),
        out_shape=jax.ShapeDtypeStruct((batch, NP), jnp.int32),
    )(X, wb, xib)

    paths = _sc_descend(c.reshape(batch * NP))
    paths = paths.transpose(0, 2, 1).reshape(batch, HEIGHT)
    root = jnp.zeros((batch, 1), dtype=jnp.int32)
    return jnp.concatenate([root, paths], axis=1)


# speedup vs baseline: 1.3062x; 1.3062x over previous
import jax
import jax.numpy as jnp
from jax.experimental import pallas as pl
from jax.experimental.pallas import tpu as pltpu

HEIGHT = 10
D = 2048
N_NODES = 2046
NP = 2048
BM = 512
OUTW = 128


def _fused_kernel(x_ref, w_ref, xi_ref, y_ref):
    f = jax.lax.dot_general(
        x_ref[...].astype(jnp.bfloat16), w_ref[...],
        (((1,), (0,)), ((), ())),
        preferred_element_type=jnp.float32)
    fb = f.astype(jnp.bfloat16)
    s = jax.lax.dot_general(
        fb, xi_ref[...], (((1,), (0,)), ((), ())),
        preferred_element_type=jnp.float32)
    g = s - jnp.concatenate([s[:, 1:], s[:, :1]], axis=1)
    y_ref[...] = jnp.zeros((BM, OUTW), jnp.int32)
    cur = jnp.zeros((BM, 1), jnp.int32)
    for h in range(HEIGHT):
        w_h = 2 << h
        off = w_h - 2
        gw = g[:, off:off + w_h]
        lane = jax.lax.broadcasted_iota(jnp.int32, (BM, w_h), 1)
        d = jnp.sum(jnp.where(lane == cur - off, gw, 0.0), axis=1,
                    keepdims=True)
        b = (d >= 0).astype(jnp.int32)
        nxt = cur + 2 - b
        y_ref[:, h + 1:h + 2] = nxt
        cur = 2 * nxt


def kernel(X, W, Xi):
    batch = X.shape[0]
    wb = W.astype(jnp.bfloat16)
    xib = jnp.pad(Xi.astype(jnp.bfloat16), ((0, 0), (0, NP - N_NODES)))

    y = pl.pallas_call(
        _fused_kernel,
        grid=(batch // BM,),
        in_specs=[
            pl.BlockSpec((BM, D), lambda i: (i, 0)),
            pl.BlockSpec((D, D), lambda i: (0, 0)),
            pl.BlockSpec((D, NP), lambda i: (0, 0)),
        ],
        out_specs=pl.BlockSpec((BM, OUTW), lambda i: (i, 0)),
        out_shape=jax.ShapeDtypeStruct((batch, OUTW), jnp.int32),
    )(X, wb, xib)

    return y[:, :HEIGHT + 1]
